# baseline (speedup 1.0000x reference)
import jax
import jax.numpy as jnp
from jax import lax
from jax.experimental import pallas as pl
from jax.experimental.pallas import tpu as pltpu
from jax.experimental.pallas import tpu_sc as plsc

NX, NY, C = 432, 496, 64
B = 4
P = 40000
NYNX = NY * NX
G = B * NYNX

NCORE, NSUB, L = 2, 16, 16
NTILE = NCORE * NSUB

PA = 1280
NSLOT = P // PA + 1
PB = 2560
RC = G // NSUB
KROWS = 8
KCH = KROWS * NX
CPS = C // NCORE


def _iota():
    return lax.iota(jnp.int32, L)


def _body(feats_hbm, coords_hbm, out_hbm, feat_t_hbm, inv_hbm, gidx_sh):
    cid = lax.axis_index("c")
    sid = lax.axis_index("s")
    iota = _iota()

    def phase_a(featbuf, cb0, cb1, sa0, sa1):
        cbase = cid * CPS

        def do_slot(t):
            start = pl.multiple_of(
                jnp.where(t < NSLOT - 1, t * PA, P - PA), 64)
            pltpu.sync_copy(feats_hbm.at[pl.ds(start * C, PA * C)], featbuf)

            def col_gather(c, cb):
                for j0 in range(0, PA // L, 8):
                    vals = [
                        plsc.load_gather(
                            featbuf, [((j0 + j) * L + iota) * C + c])
                        for j in range(8)
                    ]
                    for j in range(8):
                        cb[pl.ds((j0 + j) * L, L)] = vals[j]

            def emit_col(c, cb, sa):
                pltpu.async_copy(
                    cb, feat_t_hbm.at[pl.ds(c * P + start, PA)], sa)

            def wait_col(cb, sa):
                pltpu.make_async_copy(
                    cb, feat_t_hbm.at[pl.ds(0, PA)], sa).wait()

            def col_loop(c2, _):
                c0 = cbase + 2 * c2

                @pl.when(c2 > 0)
                def _():
                    wait_col(cb0, sa0)

                col_gather(c0, cb0)
                emit_col(c0, cb0, sa0)

                @pl.when(c2 > 0)
                def _():
                    wait_col(cb1, sa1)

                col_gather(c0 + 1, cb1)
                emit_col(c0 + 1, cb1, sa1)
                return 0

            lax.fori_loop(0, CPS // 2, col_loop, 0)
            wait_col(cb0, sa0)
            wait_col(cb1, sa1)

        do_slot(2 * sid)
        do_slot(2 * sid + 1)

    pl.run_scoped(
        phase_a,
        pltpu.VMEM((PA * C,), jnp.float32),
        pltpu.VMEM((PA,), jnp.float32),
        pltpu.VMEM((PA,), jnp.float32),
        pltpu.SemaphoreType.DMA,
        pltpu.SemaphoreType.DMA,
    )

    def phase_b(coordbuf, gidxbuf):
        start = pl.multiple_of(jnp.where(sid < NSUB - 1, sid * PB, P - PB), 64)
        pltpu.sync_copy(coords_hbm.at[pl.ds(start * 4, PB * 4)], coordbuf)

        def j_loop(j, _):
            row4 = (j * L + iota) * 4
            xi = plsc.load_gather(coordbuf, [row4])
            yi = plsc.load_gather(coordbuf, [row4 + 1])
            bi = plsc.load_gather(coordbuf, [row4 + 3])
            g = bi * NYNX + yi * NX + xi
            gidxbuf[pl.ds(j * L, L)] = g
            return 0

        lax.fori_loop(0, PB // L, j_loop, 0)
        pltpu.sync_copy(gidxbuf, gidx_sh.at[pl.ds(start, PB)])

    pl.run_scoped(
        phase_b,
        pltpu.VMEM((PB * 4,), jnp.int32),
        pltpu.VMEM((PB,), jnp.int32),
    )
    plsc.subcore_barrier()

    def phase_c(invbuf, gbuf):
        base = sid * RC

        def z_loop(z, _):
            invbuf[pl.ds(z * L, L)] = jnp.zeros((L,), jnp.int32)
            return 0

        lax.fori_loop(0, RC // L, z_loop, 0)
        pltpu.sync_copy(gidx_sh, gbuf)

        def j_loop(j, _):
            g = gbuf[pl.ds(j * L, L)]
            pv = j * L + iota + 8
            local = g - base
            inb = (local >= 0) & (local < RC)
            lc = jnp.clip(local, 0, RC - 1)
            _, last = plsc.scan_count(lc, mask=inb)
            plsc.store_scatter(invbuf, [lc], pv, mask=inb & last)
            return 0

        lax.fori_loop(0, P // L, j_loop, 0)
        pltpu.sync_copy(invbuf, inv_hbm.at[pl.ds(base, RC)])

    pl.run_scoped(
        phase_c,
        pltpu.VMEM((RC,), jnp.int32),
        pltpu.VMEM((P,), jnp.int32),
    )
    plsc.subcore_barrier()

    def phase_d(col0, col1, ib0, ib1, ob00, ob01, ob10, ob11,
                si0, si1, so00, so01, so10, so11):
        nkc = NY // KROWS
        nchk = B * nkc

        ch0 = cid * CPS + 2 * sid
        ch1 = ch0 + 1
        col0[pl.ds(0, L)] = jnp.zeros((L,), jnp.float32)
        col1[pl.ds(0, L)] = jnp.zeros((L,), jnp.float32)
        pltpu.sync_copy(feat_t_hbm.at[pl.ds(ch0 * P, P)], col0.at[pl.ds(8, P)])
        pltpu.sync_copy(feat_t_hbm.at[pl.ds(ch1 * P, P)], col1.at[pl.ds(8, P)])

        def wait_ib(ib, si):
            pltpu.make_async_copy(inv_hbm.at[pl.ds(0, KCH)], ib, si).wait()

        def wait_ob(ob, so):
            pltpu.make_async_copy(
                ob, out_hbm.at[0, 0, pl.ds(0, KROWS), :], so).wait()

        def fetch_inv(c, ib, si):
            pltpu.async_copy(inv_hbm.at[pl.ds(c * KCH, KCH)], ib, si)

        def gather_chunk(ib, oba, obb):
            grp = 9

            def row(g, _):
                base = g * NX
                for t0 in range(0, NX // L, grp):
                    ivs = [ib[pl.ds(base + (t0 + t) * L, L)]
                           for t in range(grp)]
                    va = [plsc.load_gather(col0, [iv]) for iv in ivs]
                    vb = [plsc.load_gather(col1, [iv]) for iv in ivs]
                    for t in range(grp):
                        oba[g, pl.ds((t0 + t) * L, L)] = va[t]
                        obb[g, pl.ds((t0 + t) * L, L)] = vb[t]
                return 0

            lax.fori_loop(0, KROWS, row, 0)

        def emit(c, ch, ob, so):
            dst = out_hbm.at[c // nkc, ch, pl.ds((c % nkc) * KROWS, KROWS), :]
            pltpu.async_copy(ob, dst, so)

        fetch_inv(0, ib0, si0)

        def pair_loop(p, _):
            c0 = 2 * p
            c1 = 2 * p + 1
            wait_ib(ib0, si0)
            fetch_inv(c1, ib1, si1)

            @pl.when(p > 0)
            def _():
                wait_ob(ob00, so00)
                wait_ob(ob01, so01)

            gather_chunk(ib0, ob00, ob01)
            emit(c0, ch0, ob00, so00)
            emit(c0, ch1, ob01, so01)

            wait_ib(ib1, si1)

            @pl.when(p < nchk // 2 - 1)
            def _():
                fetch_inv(c1 + 1, ib0, si0)

            @pl.when(p > 0)
            def _():
                wait_ob(ob10, so10)
                wait_ob(ob11, so11)

            gather_chunk(ib1, ob10, ob11)
            emit(c1, ch0, ob10, so10)
            emit(c1, ch1, ob11, so11)
            return 0

        lax.fori_loop(0, nchk // 2, pair_loop, 0)
        wait_ob(ob00, so00)
        wait_ob(ob01, so01)
        wait_ob(ob10, so10)
        wait_ob(ob11, so11)

    pl.run_scoped(
        phase_d,
        pltpu.VMEM((P + 8,), jnp.float32),
        pltpu.VMEM((P + 8,), jnp.float32),
        pltpu.VMEM((KCH,), jnp.int32),
        pltpu.VMEM((KCH,), jnp.int32),
        pltpu.VMEM((KROWS, NX), jnp.float32),
        pltpu.VMEM((KROWS, NX), jnp.float32),
        pltpu.VMEM((KROWS, NX), jnp.float32),
        pltpu.VMEM((KROWS, NX), jnp.float32),
        pltpu.SemaphoreType.DMA,
        pltpu.SemaphoreType.DMA,
        pltpu.SemaphoreType.DMA,
        pltpu.SemaphoreType.DMA,
        pltpu.SemaphoreType.DMA,
        pltpu.SemaphoreType.DMA,
    )


def kernel(pillar_features, voxel_coords):
    coords_flat = voxel_coords.astype(jnp.int32).reshape(-1)
    feats_flat = pillar_features.reshape(-1)
    mesh = plsc.VectorSubcoreMesh(core_axis_name="c", subcore_axis_name="s")

    k = pl.kernel(
        _body,
        out_type=(
            jax.ShapeDtypeStruct((B, C, NY, NX), jnp.float32),
            jax.ShapeDtypeStruct((C * P,), jnp.float32),
            jax.ShapeDtypeStruct((G,), jnp.int32),
        ),
        mesh=mesh,
        compiler_params=pltpu.CompilerParams(needs_layout_passes=False),
        scratch_types=[pltpu.VMEM_SHARED((P,), jnp.int32)],
    )
    out, _, _ = k(feats_flat, coords_flat)
    return out


# speedup vs baseline: 1.0140x; 1.0140x over previous
import functools

import jax
import jax.numpy as jnp
from jax import lax
from jax.experimental import pallas as pl
from jax.experimental.pallas import tpu as pltpu
from jax.experimental.pallas import tpu_sc as plsc

NX, NY, C = 432, 496, 64
B = 4
P = 40000
NYNX = NY * NX
G = B * NYNX

NCORE, NSUB, L = 2, 16, 16
NTILE = NCORE * NSUB

PA = 1280
PB = 2560
RC = G // NTILE
KROWS = 8
KCH = KROWS * NX


def _iota():
    return lax.iota(jnp.int32, L)


def _k1_body(feats_hbm, coords_hbm, feat_t_hbm, inv_hbm, gidx_sh):
    cid = lax.axis_index("c")
    sid = lax.axis_index("s")
    wid = sid * NCORE + cid
    iota = _iota()

    def phase_a(featbuf, cb0, cb1, sa0, sa1):
        start = pl.multiple_of(jnp.where(wid < NTILE - 1, wid * PA, P - PA), 64)
        pltpu.sync_copy(feats_hbm.at[pl.ds(start * C, PA * C)], featbuf)

        def col_gather(c, cb):
            for j0 in range(0, PA // L, 8):
                vals = [
                    plsc.load_gather(featbuf, [((j0 + j) * L + iota) * C + c])
                    for j in range(8)
                ]
                for j in range(8):
                    cb[pl.ds((j0 + j) * L, L)] = vals[j]

        def emit_col(c, cb, sa):
            pltpu.async_copy(cb, feat_t_hbm.at[pl.ds(c * P + start, PA)], sa)

        def wait_col(cb, sa):
            pltpu.make_async_copy(
                cb, feat_t_hbm.at[pl.ds(0, PA)], sa).wait()

        def col_loop(c2, _):
            c0 = 2 * c2

            @pl.when(c2 > 0)
            def _():
                wait_col(cb0, sa0)

            col_gather(c0, cb0)
            emit_col(c0, cb0, sa0)

            @pl.when(c2 > 0)
            def _():
                wait_col(cb1, sa1)

            col_gather(c0 + 1, cb1)
            emit_col(c0 + 1, cb1, sa1)
            return 0

        lax.fori_loop(0, C // 2, col_loop, 0)
        wait_col(cb0, sa0)
        wait_col(cb1, sa1)

    pl.run_scoped(
        phase_a,
        pltpu.VMEM((PA * C,), jnp.float32),
        pltpu.VMEM((PA,), jnp.float32),
        pltpu.VMEM((PA,), jnp.float32),
        pltpu.SemaphoreType.DMA,
        pltpu.SemaphoreType.DMA,
    )

    def phase_b(coordbuf, gidxbuf):
        start = pl.multiple_of(jnp.where(sid < NSUB - 1, sid * PB, P - PB), 64)
        pltpu.sync_copy(coords_hbm.at[pl.ds(start * 4, PB * 4)], coordbuf)

        def j_loop(j, _):
            row4 = (j * L + iota) * 4
            xi = plsc.load_gather(coordbuf, [row4])
            yi = plsc.load_gather(coordbuf, [row4 + 1])
            bi = plsc.load_gather(coordbuf, [row4 + 3])
            g = bi * NYNX + yi * NX + xi
            gidxbuf[pl.ds(j * L, L)] = g
            return 0

        lax.fori_loop(0, PB // L, j_loop, 0)
        pltpu.sync_copy(gidxbuf, gidx_sh.at[pl.ds(start, PB)])

    pl.run_scoped(
        phase_b,
        pltpu.VMEM((PB * 4,), jnp.int32),
        pltpu.VMEM((PB,), jnp.int32),
    )
    plsc.subcore_barrier()

    def phase_c(invbuf, gbuf):
        base = wid * RC

        def z_loop(z, _):
            invbuf[pl.ds(z * L, L)] = jnp.zeros((L,), jnp.int32)
            return 0

        lax.fori_loop(0, RC // L, z_loop, 0)
        pltpu.sync_copy(gidx_sh, gbuf)

        def j_loop(j, _):
            g = gbuf[pl.ds(j * L, L)]
            pv = j * L + iota + 8
            local = g - base
            inb = (local >= 0) & (local < RC)
            lc = jnp.clip(local, 0, RC - 1)
            _, last = plsc.scan_count(lc, mask=inb)
            plsc.store_scatter(invbuf, [lc], pv, mask=inb & last)
            return 0

        lax.fori_loop(0, P // L, j_loop, 0)
        pltpu.sync_copy(invbuf, inv_hbm.at[pl.ds(base, RC)])

    pl.run_scoped(
        phase_c,
        pltpu.VMEM((RC,), jnp.int32),
        pltpu.VMEM((P,), jnp.int32),
    )


def _k2_body(feat_t_hbm, inv_hbm, out_hbm,
             col0, col1, ib0, ib1, ob00, ob01, ob10, ob11,
             si0, si1, so00, so01, so10, so11):
    cid = lax.axis_index("c")
    sid = lax.axis_index("s")
    wid = sid * NCORE + cid

    nkc = NY // KROWS
    nchk = B * nkc

    ch0 = wid * 2
    ch1 = ch0 + 1
    col0[pl.ds(0, L)] = jnp.zeros((L,), jnp.float32)
    col1[pl.ds(0, L)] = jnp.zeros((L,), jnp.float32)
    pltpu.sync_copy(feat_t_hbm.at[pl.ds(ch0 * P, P)], col0.at[pl.ds(8, P)])
    pltpu.sync_copy(feat_t_hbm.at[pl.ds(ch1 * P, P)], col1.at[pl.ds(8, P)])

    def wait_ib(ib, si):
        pltpu.make_async_copy(inv_hbm.at[pl.ds(0, KCH)], ib, si).wait()

    def wait_ob(ob, so):
        pltpu.make_async_copy(
            ob, out_hbm.at[0, 0, pl.ds(0, KROWS), :], so).wait()

    def fetch_inv(c, ib, si):
        pltpu.async_copy(inv_hbm.at[pl.ds(c * KCH, KCH)], ib, si)

    def gather_chunk(ib, oba, obb):
        grp = 9

        def row(g, _):
            base = g * NX
            for t0 in range(0, NX // L, grp):
                ivs = [ib[pl.ds(base + (t0 + t) * L, L)] for t in range(grp)]
                va = [plsc.load_gather(col0, [iv]) for iv in ivs]
                vb = [plsc.load_gather(col1, [iv]) for iv in ivs]
                for t in range(grp):
                    oba[g, pl.ds((t0 + t) * L, L)] = va[t]
                    obb[g, pl.ds((t0 + t) * L, L)] = vb[t]
            return 0

        lax.fori_loop(0, KROWS, row, 0)

    def emit(c, ch, ob, so):
        dst = out_hbm.at[c // nkc, ch, pl.ds((c % nkc) * KROWS, KROWS), :]
        pltpu.async_copy(ob, dst, so)

    fetch_inv(0, ib0, si0)

    def pair_loop(p, _):
        c0 = 2 * p
        c1 = 2 * p + 1
        wait_ib(ib0, si0)
        fetch_inv(c1, ib1, si1)

        @pl.when(p > 0)
        def _():
            wait_ob(ob00, so00)
            wait_ob(ob01, so01)

        gather_chunk(ib0, ob00, ob01)
        emit(c0, ch0, ob00, so00)
        emit(c0, ch1, ob01, so01)

        wait_ib(ib1, si1)

        @pl.when(p < nchk // 2 - 1)
        def _():
            fetch_inv(c1 + 1, ib0, si0)

        @pl.when(p > 0)
        def _():
            wait_ob(ob10, so10)
            wait_ob(ob11, so11)

        gather_chunk(ib1, ob10, ob11)
        emit(c1, ch0, ob10, so10)
        emit(c1, ch1, ob11, so11)
        return 0

    lax.fori_loop(0, nchk // 2, pair_loop, 0)
    wait_ob(ob00, so00)
    wait_ob(ob01, so01)
    wait_ob(ob10, so10)
    wait_ob(ob11, so11)


def kernel(pillar_features, voxel_coords):
    coords_flat = voxel_coords.astype(jnp.int32).reshape(-1)
    feats_flat = pillar_features.reshape(-1)
    mesh = plsc.VectorSubcoreMesh(core_axis_name="c", subcore_axis_name="s")

    k1 = pl.kernel(
        _k1_body,
        out_type=(
            jax.ShapeDtypeStruct((C * P,), jnp.float32),
            jax.ShapeDtypeStruct((G,), jnp.int32),
        ),
        mesh=mesh,
        compiler_params=pltpu.CompilerParams(needs_layout_passes=False),
        scratch_types=[pltpu.VMEM_SHARED((P,), jnp.int32)],
    )
    feat_t, inv = k1(feats_flat, coords_flat)

    k2 = pl.kernel(
        _k2_body,
        out_type=jax.ShapeDtypeStruct((B, C, NY, NX), jnp.float32),
        mesh=mesh,
        compiler_params=pltpu.CompilerParams(needs_layout_passes=False),
        scratch_types=[
            pltpu.VMEM((P + 8,), jnp.float32),
            pltpu.VMEM((P + 8,), jnp.float32),
            pltpu.VMEM((KCH,), jnp.int32),
            pltpu.VMEM((KCH,), jnp.int32),
            pltpu.VMEM((KROWS, NX), jnp.float32),
            pltpu.VMEM((KROWS, NX), jnp.float32),
            pltpu.VMEM((KROWS, NX), jnp.float32),
            pltpu.VMEM((KROWS, NX), jnp.float32),
            pltpu.SemaphoreType.DMA,
            pltpu.SemaphoreType.DMA,
            pltpu.SemaphoreType.DMA,
            pltpu.SemaphoreType.DMA,
            pltpu.SemaphoreType.DMA,
            pltpu.SemaphoreType.DMA,
        ],
    )
    return k2(feat_t, inv)
